# baseline (speedup 1.0000x reference)
import functools

import jax
import jax.numpy as jnp
from jax import lax
from jax.experimental import pallas as pl
from jax.experimental.pallas import tpu as pltpu
from jax.experimental.pallas import tpu_sc as plsc

BATCH = 16384
N_IN = 16
N_RULES = 64
NUM_CORES = 2
NUM_SUBCORES = 16
NUM_WORKERS = NUM_CORES * NUM_SUBCORES
ROWS_PER_WORKER = BATCH // NUM_WORKERS
LANES = 16
ROWS_PER_BLOCK = 16
BLOCKS = ROWS_PER_WORKER // ROWS_PER_BLOCK

IN_WORDS = ROWS_PER_WORKER * N_IN
OUT_WORDS = ROWS_PER_WORKER * N_RULES

_GATHER_DNUMS = lax.GatherDimensionNumbers(
    offset_dims=(), collapsed_slice_dims=(0,), start_index_map=(0,))


def _perm(v, idx):
    return lax.gather(v, idx[:, None], _GATHER_DNUMS, slice_sizes=(1,),
                      mode=lax.GatherScatterMode.PROMISE_IN_BOUNDS)


def _body(x_hbm, out_hbm, in_v, out_v):
    wid = lax.axis_index("s") * NUM_CORES + lax.axis_index("c")

    pltpu.sync_copy(x_hbm.at[pl.ds(wid * IN_WORDS, IN_WORDS)], in_v)

    iota = lax.iota(jnp.int32, LANES)
    idx_c = 8 + jnp.bitwise_and(iota, 7)
    idx_a = [2 * k + jnp.right_shift(iota, 3) for k in range(4)]

    def block(t, carry):
        row0 = t * ROWS_PER_BLOCK
        for r in range(ROWS_PER_BLOCK):
            row = row0 + r
            v = in_v[pl.ds(row * N_IN, LANES)]
            c = _perm(v, idx_c)
            for k in range(4):
                a = _perm(v, idx_a[k])
                out_v[pl.ds(row * N_RULES + 16 * k, 16)] = jnp.minimum(a, c)
        return carry

    lax.fori_loop(0, BLOCKS, block, 0)

    pltpu.sync_copy(out_v, out_hbm.at[pl.ds(wid * OUT_WORDS, OUT_WORDS)])


@functools.partial(jax.jit, static_argnames=())
def _run(x_flat):
    mesh = plsc.VectorSubcoreMesh(
        core_axis_name="c", subcore_axis_name="s",
        num_cores=NUM_CORES, num_subcores=NUM_SUBCORES,
    )
    k = pl.kernel(
        _body,
        out_type=jax.ShapeDtypeStruct((BATCH * N_RULES,), jnp.float32),
        mesh=mesh,
        scratch_types=[
            pltpu.VMEM((IN_WORDS,), jnp.float32),
            pltpu.VMEM((OUT_WORDS,), jnp.float32),
        ],
        compiler_params=pltpu.CompilerParams(needs_layout_passes=False),
    )
    return k(x_flat).reshape(BATCH, N_RULES)


def kernel(x):
    return _run(x.reshape(BATCH * N_IN))


# speedup vs baseline: 1.0012x; 1.0012x over previous
import functools

import jax
import jax.numpy as jnp
from jax import lax
from jax.experimental import pallas as pl
from jax.experimental.pallas import tpu as pltpu
from jax.experimental.pallas import tpu_sc as plsc

BATCH = 16384
N_IN = 16
N_RULES = 64
NUM_CORES = 2
NUM_SUBCORES = 16
NUM_WORKERS = NUM_CORES * NUM_SUBCORES
ROWS_PER_WORKER = BATCH // NUM_WORKERS
LANES = 16
ROWS_PER_BLOCK = 16
BLOCKS = ROWS_PER_WORKER // ROWS_PER_BLOCK

IN_VROWS = ROWS_PER_WORKER * N_IN // 128
OUT_VROWS = ROWS_PER_WORKER * N_RULES // 128

_GATHER_DNUMS = lax.GatherDimensionNumbers(
    offset_dims=(), collapsed_slice_dims=(0,), start_index_map=(0,))


def _perm(v, idx):
    return lax.gather(v, idx[:, None], _GATHER_DNUMS, slice_sizes=(1,),
                      mode=lax.GatherScatterMode.PROMISE_IN_BOUNDS)


def _body(x_hbm, out_hbm, in_v, out_v):
    wid = lax.axis_index("s") * NUM_CORES + lax.axis_index("c")

    pltpu.sync_copy(x_hbm.at[pl.ds(wid * IN_VROWS, IN_VROWS)], in_v)

    iota = lax.iota(jnp.int32, LANES)
    idx_c = 8 + jnp.bitwise_and(iota, 7)
    idx_a = [2 * k + jnp.right_shift(iota, 3) for k in range(4)]

    def block(t, carry):
        for r in range(ROWS_PER_BLOCK):
            vrow = 2 * t + (r >> 3)
            v = in_v[vrow, pl.ds((r & 7) * LANES, LANES)]
            c = _perm(v, idx_c)
            for k in range(4):
                a = _perm(v, idx_a[k])
                q = 4 * r + k
                out_v[8 * t + (q >> 3), pl.ds((q & 7) * LANES, LANES)] = (
                    jnp.minimum(a, c))
        return carry

    lax.fori_loop(0, BLOCKS, block, 0)

    pltpu.sync_copy(out_v, out_hbm.at[pl.ds(wid * OUT_VROWS, OUT_VROWS)])


@functools.partial(jax.jit, static_argnames=())
def _run(x2):
    mesh = plsc.VectorSubcoreMesh(
        core_axis_name="c", subcore_axis_name="s",
        num_cores=NUM_CORES, num_subcores=NUM_SUBCORES,
    )
    k = pl.kernel(
        _body,
        out_type=jax.ShapeDtypeStruct((BATCH * N_RULES // 128, 128),
                                      jnp.float32),
        mesh=mesh,
        scratch_types=[
            pltpu.VMEM((IN_VROWS, 128), jnp.float32),
            pltpu.VMEM((OUT_VROWS, 128), jnp.float32),
        ],
        compiler_params=pltpu.CompilerParams(needs_layout_passes=False),
    )
    return k(x2)


def kernel(x):
    x2 = x.reshape(BATCH * N_IN // 128, 128)
    return _run(x2).reshape(BATCH, N_RULES)


# speedup vs baseline: 4.1509x; 4.1461x over previous
import functools

import jax
import jax.numpy as jnp
from jax import lax
from jax.experimental import pallas as pl
from jax.experimental.pallas import tpu as pltpu
from jax.experimental.pallas import tpu_sc as plsc

BATCH = 16384
N_IN = 16
N_RULES = 64
NUM_CORES = 2
NUM_SUBCORES = 16
NUM_WORKERS = NUM_CORES * NUM_SUBCORES
COLS_PER_WORKER = BATCH // NUM_WORKERS
LANES = 16
GROUPS = COLS_PER_WORKER // LANES


def _body(xt_hbm, outt_hbm, in_v, out_v):
    wid = lax.axis_index("s") * NUM_CORES + lax.axis_index("c")
    base = wid * COLS_PER_WORKER

    pltpu.sync_copy(xt_hbm.at[:, pl.ds(base, COLS_PER_WORKER)], in_v)

    def group(g, carry):
        col = g * LANES
        vals = [in_v[p, pl.ds(col, LANES)] for p in range(N_IN)]
        a, c = vals[:8], vals[8:]
        for i in range(8):
            for j in range(8):
                out_v[i * 8 + j, pl.ds(col, LANES)] = jnp.minimum(a[i], c[j])
        return carry

    lax.fori_loop(0, GROUPS, group, 0)

    pltpu.sync_copy(out_v, outt_hbm.at[:, pl.ds(base, COLS_PER_WORKER)])


@functools.partial(jax.jit, static_argnames=())
def _run(xt):
    mesh = plsc.VectorSubcoreMesh(
        core_axis_name="c", subcore_axis_name="s",
        num_cores=NUM_CORES, num_subcores=NUM_SUBCORES,
    )
    k = pl.kernel(
        _body,
        out_type=jax.ShapeDtypeStruct((N_RULES, BATCH), jnp.float32),
        mesh=mesh,
        scratch_types=[
            pltpu.VMEM((N_IN, COLS_PER_WORKER), jnp.float32),
            pltpu.VMEM((N_RULES, COLS_PER_WORKER), jnp.float32),
        ],
        compiler_params=pltpu.CompilerParams(needs_layout_passes=False),
    )
    return k(xt)


def kernel(x):
    xt = x.transpose(1, 2, 0).reshape(N_IN, BATCH)
    return _run(xt).T
